# baseline (speedup 1.0000x reference)
import functools
import jax
import jax.numpy as jnp
from jax import lax
from jax.experimental import pallas as pl
from jax.experimental.pallas import tpu as pltpu
from jax.experimental.pallas import tpu_sc as plsc

_BATCH = 16384
_DM, _DS, _DD, _DH = 128, 64, 32, 128
_DOUT = _DM + _DS + _DD + _DH
_NC, _NS, _L = 2, 16, 16
_NW = _NC * _NS
_CHUNK = 128
_NCHUNK = _BATCH // (_NW * _CHUNK)

_mesh = plsc.VectorSubcoreMesh(core_axis_name="c", subcore_axis_name="s")


@functools.partial(
    pl.kernel,
    mesh=_mesh,
    compiler_params=pltpu.CompilerParams(use_tc_tiling_on_sc=False),
    out_type=jax.ShapeDtypeStruct((_BATCH, _DOUT), jnp.float32),
    scratch_types=[
        pltpu.VMEM((_CHUNK,), jnp.int32),
        pltpu.VMEM((_CHUNK,), jnp.int32),
        pltpu.VMEM((_CHUNK,), jnp.int32),
        pltpu.VMEM((_CHUNK,), jnp.int32),
        pltpu.VMEM((_CHUNK, _DM), jnp.float32),
        pltpu.VMEM((_CHUNK, _DS), jnp.float32),
        pltpu.VMEM((_CHUNK, _DD), jnp.float32),
        pltpu.VMEM((_CHUNK, _DH), jnp.float32),
        pltpu.SemaphoreType.DMA,
    ],
)
def _embedder(month_hbm, day_hbm, hh_hbm, wm, ws, wd, wh, out,
              midx, sidx, didx, hidx, mrows, srows, drows, hrows, sem):
    wid = lax.axis_index("s") * _NC + lax.axis_index("c")
    for k in range(_NCHUNK):
        row = wid * _NCHUNK + k
        off = row * _CHUNK
        pltpu.sync_copy(month_hbm.at[row], midx)
        pltpu.sync_copy(day_hbm.at[row], didx)
        pltpu.sync_copy(hh_hbm.at[row], hidx)
        one = jnp.full((_L,), 1, jnp.int32)
        three = jnp.full((_L,), 3, jnp.int32)
        four = jnp.full((_L,), 4, jnp.int32)
        for j in range(_CHUNK // _L):
            m = midx[pl.ds(j * _L, _L)]
            sidx[pl.ds(j * _L, _L)] = lax.rem(lax.div(lax.add(m, one), three), four)
        cm = pltpu.async_copy(wm.at[midx], mrows, sem)
        cs = pltpu.async_copy(ws.at[sidx], srows, sem)
        cd = pltpu.async_copy(wd.at[didx], drows, sem)
        ch = pltpu.async_copy(wh.at[hidx], hrows, sem)
        cm.wait()
        cs.wait()
        cd.wait()
        ch.wait()
        pltpu.sync_copy(mrows, out.at[pl.ds(off, _CHUNK), pl.ds(0, _DM)])
        pltpu.sync_copy(srows, out.at[pl.ds(off, _CHUNK), pl.ds(_DM, _DS)])
        pltpu.sync_copy(drows, out.at[pl.ds(off, _CHUNK), pl.ds(_DM + _DS, _DD)])
        pltpu.sync_copy(hrows, out.at[pl.ds(off, _CHUNK), pl.ds(_DM + _DS + _DD, _DH)])


def kernel(month, day_type, household_id, W_month, W_season, W_day_type, W_household):
    m2 = month.astype(jnp.int32).reshape(_NW * _NCHUNK, _CHUNK)
    d2 = day_type.astype(jnp.int32).reshape(_NW * _NCHUNK, _CHUNK)
    h2 = household_id.astype(jnp.int32).reshape(_NW * _NCHUNK, _CHUNK)
    return _embedder(m2, d2, h2, W_month, W_season, W_day_type, W_household)
